# baseline (speedup 1.0000x reference)
import functools

import jax
import jax.numpy as jnp
from jax import lax
from jax.experimental import pallas as pl
from jax.experimental.pallas import tpu as pltpu
from jax.experimental.pallas import tpu_sc as plsc

_SAMPLE_TO = 1000
_LANE_CHUNK = 128


def _build_gather(total: int, n_elems: int):
  info = plsc.get_sparse_core_info()
  nw = info.num_cores * info.num_subcores
  n_rows = n_elems // _LANE_CHUNK
  rows_per_w = n_rows // nw

  mesh = plsc.VectorSubcoreMesh(core_axis_name="c", subcore_axis_name="s")

  @functools.partial(
      pl.kernel,
      mesh=mesh,
      out_type=jax.ShapeDtypeStruct((nw, rows_per_w, _LANE_CHUNK),
                                    jnp.float32),
      scratch_types=[
          pltpu.VMEM((rows_per_w, _LANE_CHUNK), jnp.int32),
          pltpu.VMEM((rows_per_w, _LANE_CHUNK), jnp.float32),
          pltpu.SemaphoreType.DMA,
      ],
  )
  def gather_kernel(flat_hbm, idx_hbm, out_hbm, idx_v, vals_v, sem):
    wid = lax.axis_index("s") * info.num_cores + lax.axis_index("c")
    pltpu.sync_copy(idx_hbm.at[wid], idx_v)

    def fire(j, _):
      pltpu.async_copy(flat_hbm.at[idx_v.at[j]], vals_v.at[j], sem)
      return _

    def drain(j, _):
      pltpu.make_async_copy(flat_hbm.at[idx_v.at[j]], vals_v.at[j],
                            sem).wait()
      return _

    lax.fori_loop(0, rows_per_w, fire, None)
    lax.fori_loop(0, rows_per_w, drain, None)
    pltpu.sync_copy(vals_v, out_hbm.at[wid])

  return gather_kernel


def kernel(inputs):
  rows, k = inputs.shape
  if k <= _SAMPLE_TO:
    return inputs
  perm = jax.random.permutation(jax.random.key(42), k)
  ridxs = perm[:_SAMPLE_TO].astype(jnp.int32)
  row_base = jnp.arange(rows, dtype=jnp.int32) * k
  info = plsc.get_sparse_core_info()
  nw = info.num_cores * info.num_subcores
  idx = (row_base[:, None] + ridxs[None, :]).reshape(nw, -1, _LANE_CHUNK)
  flat = inputs.reshape(-1)
  out = _build_gather(rows * k, rows * _SAMPLE_TO)(flat, idx)
  return out.reshape(rows, _SAMPLE_TO)
